# baseline (speedup 1.0000x reference)
import functools

import jax
import jax.numpy as jnp
import numpy as np
from jax import lax
from jax.experimental import pallas as pl
from jax.experimental.pallas import tpu as pltpu
from jax.experimental.pallas import tpu_sc as plsc

N = 10000
E = 320000
D = 128
H = 128
BN_EPS = 1e-5

NC = 2
NS = 16
NW = NC * NS
CH = 128
K = 80
EPAD = NW * K * CH
NPAD = 10240
RPT = NPAD // NS

_mesh = plsc.VectorSubcoreMesh(core_axis_name="c", subcore_axis_name="s")


@functools.partial(
    pl.kernel,
    out_type=jax.ShapeDtypeStruct((2 * NPAD, D), jnp.float32),
    mesh=_mesh,
    scratch_types=[
        pltpu.VMEM((K // 2, CH), jnp.int32),
        pltpu.VMEM((K // 2, CH), jnp.int32),
        pltpu.VMEM((CH, D), jnp.float32),
        pltpu.VMEM((CH, D), jnp.float32),
        pltpu.VMEM_SHARED((NPAD, D), jnp.float32),
        pltpu.SemaphoreType.DMA,
        pltpu.SemaphoreType.DMA,
    ],
)
def _segsum_sc(x_hbm, src_hbm, dst_hbm, out_hbm,
               src_v, dst_v, rows0_v, rows1_v, acc, sem0, sem1):
    c = lax.axis_index("c")
    s = lax.axis_index("s")
    w = s * NC + c

    def zbody(r, carry):
        for q in range(D // 16):
            rows0_v[r, pl.ds(q * 16, 16)] = jnp.zeros((16,), jnp.float32)
        return carry

    lax.fori_loop(0, CH, zbody, 0)
    for t in range(RPT // CH):
        pltpu.async_copy(rows0_v, acc.at[pl.ds(s * RPT + t * CH, CH)], sem0)
    for t in range(RPT // CH):
        pltpu.make_async_copy(
            rows0_v, acc.at[pl.ds(s * RPT + t * CH, CH)], sem0).wait()
    plsc.subcore_barrier()

    K2 = K // 2
    for h in range(2):
        pltpu.sync_copy(src_hbm.at[w, pl.ds(h * K2, K2)], src_v)
        pltpu.sync_copy(dst_hbm.at[w, pl.ds(h * K2, K2)], dst_v)
        pltpu.async_copy(x_hbm.at[src_v.at[0]], rows0_v, sem0)

        def body(i, carry):
            j = 2 * i
            pltpu.make_async_copy(x_hbm.at[src_v.at[j]], rows0_v, sem0).wait()
            pltpu.async_copy(x_hbm.at[src_v.at[j + 1]], rows1_v, sem1)
            pltpu.sync_copy(rows0_v, acc.at[dst_v.at[j]], add=True)
            pltpu.make_async_copy(x_hbm.at[src_v.at[j + 1]], rows1_v, sem1).wait()

            @pl.when(j + 2 < K2)
            def _():
                pltpu.async_copy(x_hbm.at[src_v.at[j + 2]], rows0_v, sem0)

            pltpu.sync_copy(rows1_v, acc.at[dst_v.at[j + 1]], add=True)
            return carry

        lax.fori_loop(0, K2 // 2, body, 0)
    plsc.subcore_barrier()
    pltpu.sync_copy(acc.at[pl.ds(s * RPT, RPT)],
                    out_hbm.at[pl.ds(c * NPAD + s * RPT, RPT)])


BN_ROWS = 512
_GRID = NPAD // BN_ROWS


def _mlp_body(x_ref, p0_ref, p1_ref, w0_ref, b0_ref, w1_ref, b1_ref, o_ref):
    h = x_ref[...] + p0_ref[...] + p1_ref[...]
    h = jnp.dot(h, w0_ref[...], preferred_element_type=jnp.float32) + b0_ref[...]
    h = jnp.maximum(h, 0.0)
    h = jnp.dot(h, w1_ref[...], preferred_element_type=jnp.float32) + b1_ref[...]
    o_ref[...] = jnp.maximum(h, 0.0)


_mlp_call = pl.pallas_call(
    _mlp_body,
    grid=(_GRID,),
    in_specs=[
        pl.BlockSpec((BN_ROWS, D), lambda i: (i, 0)),
        pl.BlockSpec((BN_ROWS, D), lambda i: (i, 0)),
        pl.BlockSpec((BN_ROWS, D), lambda i: (i + _GRID, 0)),
        pl.BlockSpec((D, H), lambda i: (0, 0)),
        pl.BlockSpec((1, H), lambda i: (0, 0)),
        pl.BlockSpec((H, H), lambda i: (0, 0)),
        pl.BlockSpec((1, H), lambda i: (0, 0)),
    ],
    out_specs=pl.BlockSpec((BN_ROWS, H), lambda i: (i, 0)),
    out_shape=jax.ShapeDtypeStruct((NPAD, H), jnp.float32),
)


def _pre_head_body(x1_ref, x2_ref, wa_ref, wb_ref, b1_ref, o_ref):
    o_ref[...] = (
        jnp.dot(x1_ref[...], wa_ref[...], preferred_element_type=jnp.float32)
        + jnp.dot(x2_ref[...], wb_ref[...], preferred_element_type=jnp.float32)
        + b1_ref[...])


_pre_head_call = pl.pallas_call(
    _pre_head_body,
    grid=(_GRID,),
    in_specs=[
        pl.BlockSpec((BN_ROWS, H), lambda i: (i, 0)),
        pl.BlockSpec((BN_ROWS, H), lambda i: (i, 0)),
        pl.BlockSpec((H, 3 * H), lambda i: (0, 0)),
        pl.BlockSpec((H, 3 * H), lambda i: (0, 0)),
        pl.BlockSpec((1, 3 * H), lambda i: (0, 0)),
    ],
    out_specs=pl.BlockSpec((BN_ROWS, 3 * H), lambda i: (i, 0)),
    out_shape=jax.ShapeDtypeStruct((NPAD, 3 * H), jnp.float32),
)


def _head_body(y12_ref, x3_ref, wc_ref,
               w2_ref, b2_ref, wo_ref, bo_ref, o_ref):
    h = y12_ref[...] + jnp.dot(x3_ref[...], wc_ref[...],
                               preferred_element_type=jnp.float32)
    h = jnp.maximum(h, 0.0)
    h = jnp.dot(h, w2_ref[...], preferred_element_type=jnp.float32) + b2_ref[...]
    h = jnp.maximum(h, 0.0)
    o = jnp.dot(h, wo_ref[...], preferred_element_type=jnp.float32) + bo_ref[...]
    o_ref[...] = jnp.maximum(o, 0.0)


_head_call = pl.pallas_call(
    _head_body,
    grid=(_GRID,),
    in_specs=[
        pl.BlockSpec((BN_ROWS, 3 * H), lambda i: (i, 0)),
        pl.BlockSpec((BN_ROWS, H), lambda i: (i, 0)),
        pl.BlockSpec((H, 3 * H), lambda i: (0, 0)),
        pl.BlockSpec((3 * H, 3 * H), lambda i: (0, 0)),
        pl.BlockSpec((1, 3 * H), lambda i: (0, 0)),
        pl.BlockSpec((3 * H, 128), lambda i: (0, 0)),
        pl.BlockSpec((1, 128), lambda i: (0, 0)),
    ],
    out_specs=pl.BlockSpec((BN_ROWS, 128), lambda i: (i, 0)),
    out_shape=jax.ShapeDtypeStruct((NPAD, 128), jnp.float32),
)


def _fold_bn(Wl, bl, g, be):
    sc = g * (1.0 / np.sqrt(1.0 + BN_EPS))
    return Wl * sc[None, :], bl * sc + be


def kernel(x, edge_index, batch,
           c1_W0, c1_b0, c1_g0, c1_be0, c1_W1, c1_b1, c1_g1, c1_be1,
           c2_W0, c2_b0, c2_g0, c2_be0, c2_W1, c2_b1, c2_g1, c2_be1,
           c3_W0, c3_b0, c3_g0, c3_be0, c3_W1, c3_b1, c3_g1, c3_be1,
           lin1_W, lin1_b, lin2_W, lin2_b, out_W, out_b):
    del batch
    layers = []
    for (W0, b0, g0, be0, W1, b1, g1, be1) in (
        (c1_W0, c1_b0, c1_g0, c1_be0, c1_W1, c1_b1, c1_g1, c1_be1),
        (c2_W0, c2_b0, c2_g0, c2_be0, c2_W1, c2_b1, c2_g1, c2_be1),
        (c3_W0, c3_b0, c3_g0, c3_be0, c3_W1, c3_b1, c3_g1, c3_be1),
    ):
        W0f, b0f = _fold_bn(W0, b0, g0, be0)
        W1f, b1f = _fold_bn(W1, b1, g1, be1)
        layers.append((W0f, b0f.reshape(1, H), W1f, b1f.reshape(1, H)))

    x_pad = jnp.pad(x, ((0, NPAD - N), (0, 0)))
    junk = N + jnp.arange(EPAD - E, dtype=jnp.int32) % (NPAD - N)
    src = jnp.concatenate([edge_index[0], junk]).reshape(NW, K, CH)
    dst = jnp.concatenate([edge_index[1], junk]).reshape(NW, K, CH)

    h = x_pad
    feats = []
    for (W0f, b0f, W1f, b1f) in layers:
        parts = _segsum_sc(h, src, dst)
        h = _mlp_call(h, parts, parts, W0f, b0f, W1f, b1f)
        feats.append(h)

    x1, x2, x3 = feats
    wa, wb, wc = lin1_W[:H], lin1_W[H:2 * H], lin1_W[2 * H:]
    wo = jnp.pad(out_W, ((0, 0), (0, 127)))
    bo = jnp.pad(out_b, (0, 127)).reshape(1, 128)
    y12 = _pre_head_call(x1, x2, wa, wb, lin1_b.reshape(1, 3 * H))
    out = _head_call(y12, x3, wc,
                     lin2_W, lin2_b.reshape(1, 3 * H), wo, bo)
    return out[:N, :1]


# speedup vs baseline: 1.0020x; 1.0020x over previous
import functools

import jax
import jax.numpy as jnp
import numpy as np
from jax import lax
from jax.experimental import pallas as pl
from jax.experimental.pallas import tpu as pltpu
from jax.experimental.pallas import tpu_sc as plsc

N = 10000
E = 320000
D = 128
H = 128
BN_EPS = 1e-5

NC = 2
NS = 16
NW = NC * NS
CH = 128
K = 80
EPAD = NW * K * CH
NPAD = 10240
RPT = NPAD // NS

_mesh = plsc.VectorSubcoreMesh(core_axis_name="c", subcore_axis_name="s")


@functools.partial(
    pl.kernel,
    out_type=jax.ShapeDtypeStruct((2 * NPAD, D), jnp.float32),
    mesh=_mesh,
    scratch_types=[
        pltpu.VMEM((K // 2, CH), jnp.int32),
        pltpu.VMEM((K // 2, CH), jnp.int32),
        pltpu.VMEM((CH, D), jnp.float32),
        pltpu.VMEM((CH, D), jnp.float32),
        pltpu.VMEM_SHARED((NPAD, D), jnp.float32),
        pltpu.SemaphoreType.DMA,
        pltpu.SemaphoreType.DMA,
    ],
)
def _segsum_sc(x_hbm, src_hbm, dst_hbm, out_hbm,
               src_v, dst_v, rows0_v, rows1_v, acc, sem0, sem1):
    c = lax.axis_index("c")
    s = lax.axis_index("s")
    w = s * NC + c

    def zbody(r, carry):
        for q in range(D // 16):
            rows0_v[r, pl.ds(q * 16, 16)] = jnp.zeros((16,), jnp.float32)
        return carry

    lax.fori_loop(0, CH, zbody, 0)
    for t in range(RPT // CH):
        pltpu.async_copy(rows0_v, acc.at[pl.ds(s * RPT + t * CH, CH)], sem0)
    for t in range(RPT // CH):
        pltpu.make_async_copy(
            rows0_v, acc.at[pl.ds(s * RPT + t * CH, CH)], sem0).wait()
    plsc.subcore_barrier()

    K2 = K // 2
    for h in range(2):
        pltpu.sync_copy(src_hbm.at[w, pl.ds(h * K2, K2)], src_v)
        pltpu.sync_copy(dst_hbm.at[w, pl.ds(h * K2, K2)], dst_v)
        pltpu.async_copy(x_hbm.at[src_v.at[0]], rows0_v, sem0)

        def body(i, carry):
            j = 2 * i
            pltpu.make_async_copy(x_hbm.at[src_v.at[j]], rows0_v, sem0).wait()
            pltpu.async_copy(x_hbm.at[src_v.at[j + 1]], rows1_v, sem1)
            pltpu.sync_copy(rows0_v, acc.at[dst_v.at[j]], add=True)
            pltpu.make_async_copy(x_hbm.at[src_v.at[j + 1]], rows1_v, sem1).wait()

            @pl.when(j + 2 < K2)
            def _():
                pltpu.async_copy(x_hbm.at[src_v.at[j + 2]], rows0_v, sem0)

            pltpu.sync_copy(rows1_v, acc.at[dst_v.at[j + 1]], add=True)
            return carry

        lax.fori_loop(0, K2 // 2, body, 0)
    plsc.subcore_barrier()
    pltpu.sync_copy(acc.at[pl.ds(s * RPT, RPT)],
                    out_hbm.at[pl.ds(c * NPAD + s * RPT, RPT)])


BN_ROWS = 512
_GRID = NPAD // BN_ROWS


def _mlp_body(x_ref, p0_ref, p1_ref, w0_ref, b0_ref, w1_ref, b1_ref, o_ref):
    h = x_ref[...] + p0_ref[...] + p1_ref[...]
    h = jnp.dot(h, w0_ref[...], preferred_element_type=jnp.float32) + b0_ref[...]
    h = jnp.maximum(h, 0.0)
    h = jnp.dot(h, w1_ref[...], preferred_element_type=jnp.float32) + b1_ref[...]
    o_ref[...] = jnp.maximum(h, 0.0)


_mlp_call = pl.pallas_call(
    _mlp_body,
    grid=(_GRID,),
    in_specs=[
        pl.BlockSpec((BN_ROWS, D), lambda i: (i, 0)),
        pl.BlockSpec((BN_ROWS, D), lambda i: (i, 0)),
        pl.BlockSpec((BN_ROWS, D), lambda i: (i + _GRID, 0)),
        pl.BlockSpec((D, H), lambda i: (0, 0)),
        pl.BlockSpec((1, H), lambda i: (0, 0)),
        pl.BlockSpec((H, H), lambda i: (0, 0)),
        pl.BlockSpec((1, H), lambda i: (0, 0)),
    ],
    out_specs=pl.BlockSpec((BN_ROWS, H), lambda i: (i, 0)),
    out_shape=jax.ShapeDtypeStruct((NPAD, H), jnp.float32),
)


def _head_body(x1_ref, x2_ref, x3_ref, wa_ref, wb_ref, wc_ref, b1_ref,
               w2_ref, b2_ref, wo_ref, bo_ref, o_ref):
    h = (jnp.dot(x1_ref[...], wa_ref[...], preferred_element_type=jnp.float32)
         + jnp.dot(x2_ref[...], wb_ref[...], preferred_element_type=jnp.float32)
         + jnp.dot(x3_ref[...], wc_ref[...], preferred_element_type=jnp.float32)
         + b1_ref[...])
    h = jnp.maximum(h, 0.0)
    h = jnp.dot(h, w2_ref[...], preferred_element_type=jnp.float32) + b2_ref[...]
    h = jnp.maximum(h, 0.0)
    o = jnp.dot(h, wo_ref[...], preferred_element_type=jnp.float32) + bo_ref[...]
    o_ref[...] = jnp.maximum(o, 0.0)


_head_call = pl.pallas_call(
    _head_body,
    grid=(_GRID,),
    in_specs=[
        pl.BlockSpec((BN_ROWS, H), lambda i: (i, 0)),
        pl.BlockSpec((BN_ROWS, H), lambda i: (i, 0)),
        pl.BlockSpec((BN_ROWS, H), lambda i: (i, 0)),
        pl.BlockSpec((H, 3 * H), lambda i: (0, 0)),
        pl.BlockSpec((H, 3 * H), lambda i: (0, 0)),
        pl.BlockSpec((H, 3 * H), lambda i: (0, 0)),
        pl.BlockSpec((1, 3 * H), lambda i: (0, 0)),
        pl.BlockSpec((3 * H, 3 * H), lambda i: (0, 0)),
        pl.BlockSpec((1, 3 * H), lambda i: (0, 0)),
        pl.BlockSpec((3 * H, 128), lambda i: (0, 0)),
        pl.BlockSpec((1, 128), lambda i: (0, 0)),
    ],
    out_specs=pl.BlockSpec((BN_ROWS, 128), lambda i: (i, 0)),
    out_shape=jax.ShapeDtypeStruct((NPAD, 128), jnp.float32),
)


def _fold_bn(Wl, bl, g, be):
    sc = g * (1.0 / np.sqrt(1.0 + BN_EPS))
    return Wl * sc[None, :], bl * sc + be


def kernel(x, edge_index, batch,
           c1_W0, c1_b0, c1_g0, c1_be0, c1_W1, c1_b1, c1_g1, c1_be1,
           c2_W0, c2_b0, c2_g0, c2_be0, c2_W1, c2_b1, c2_g1, c2_be1,
           c3_W0, c3_b0, c3_g0, c3_be0, c3_W1, c3_b1, c3_g1, c3_be1,
           lin1_W, lin1_b, lin2_W, lin2_b, out_W, out_b):
    del batch
    layers = []
    for (W0, b0, g0, be0, W1, b1, g1, be1) in (
        (c1_W0, c1_b0, c1_g0, c1_be0, c1_W1, c1_b1, c1_g1, c1_be1),
        (c2_W0, c2_b0, c2_g0, c2_be0, c2_W1, c2_b1, c2_g1, c2_be1),
        (c3_W0, c3_b0, c3_g0, c3_be0, c3_W1, c3_b1, c3_g1, c3_be1),
    ):
        W0f, b0f = _fold_bn(W0, b0, g0, be0)
        W1f, b1f = _fold_bn(W1, b1, g1, be1)
        layers.append((W0f, b0f.reshape(1, H), W1f, b1f.reshape(1, H)))

    x_pad = jnp.pad(x, ((0, NPAD - N), (0, 0)))
    junk = N + jnp.arange(EPAD - E, dtype=jnp.int32) % (NPAD - N)
    src = jnp.concatenate([edge_index[0], junk]).reshape(NW, K, CH)
    dst = jnp.concatenate([edge_index[1], junk]).reshape(NW, K, CH)

    h = x_pad
    feats = []
    for (W0f, b0f, W1f, b1f) in layers:
        parts = _segsum_sc(h, src, dst)
        h = _mlp_call(h, parts, parts, W0f, b0f, W1f, b1f)
        feats.append(h)

    x1, x2, x3 = feats
    wa, wb, wc = lin1_W[:H], lin1_W[H:2 * H], lin1_W[2 * H:]
    wo = jnp.pad(out_W, ((0, 0), (0, 127)))
    bo = jnp.pad(out_b, (0, 127)).reshape(1, 128)
    out = _head_call(x1, x2, x3, wa, wb, wc, lin1_b.reshape(1, 3 * H),
                     lin2_W, lin2_b.reshape(1, 3 * H), wo, bo)
    return out[:N, :1]


# speedup vs baseline: 1.0496x; 1.0475x over previous
import functools

import jax
import jax.numpy as jnp
import numpy as np
from jax import lax
from jax.experimental import pallas as pl
from jax.experimental.pallas import tpu as pltpu
from jax.experimental.pallas import tpu_sc as plsc

N = 10000
E = 320000
D = 128
H = 128
BN_EPS = 1e-5

NC = 2
NS = 16
NW = NC * NS
CH = 128
K = 80
EPAD = NW * K * CH
NPAD = 10240
RPT = NPAD // NS

_mesh = plsc.VectorSubcoreMesh(core_axis_name="c", subcore_axis_name="s")


@functools.partial(
    pl.kernel,
    out_type=jax.ShapeDtypeStruct((2 * NPAD, D), jnp.float32),
    mesh=_mesh,
    scratch_types=[
        pltpu.VMEM((K // 2, CH), jnp.int32),
        pltpu.VMEM((K // 2, CH), jnp.int32),
        pltpu.VMEM((CH, D), jnp.float32),
        pltpu.VMEM((CH, D), jnp.float32),
        pltpu.VMEM_SHARED((NPAD, D), jnp.float32),
        pltpu.SemaphoreType.DMA,
        pltpu.SemaphoreType.DMA,
    ],
)
def _segsum_sc(x_hbm, src_hbm, dst_hbm, out_hbm,
               src_v, dst_v, rows0_v, rows1_v, acc, sem0, sem1):
    c = lax.axis_index("c")
    s = lax.axis_index("s")
    w = s * NC + c

    def zbody(r, carry):
        for q in range(D // 16):
            rows0_v[r, pl.ds(q * 16, 16)] = jnp.zeros((16,), jnp.float32)
        return carry

    lax.fori_loop(0, CH, zbody, 0)
    for t in range(RPT // CH):
        pltpu.async_copy(rows0_v, acc.at[pl.ds(s * RPT + t * CH, CH)], sem0)
    for t in range(RPT // CH):
        pltpu.make_async_copy(
            rows0_v, acc.at[pl.ds(s * RPT + t * CH, CH)], sem0).wait()
    plsc.subcore_barrier()

    K2 = K // 2
    for h in range(2):
        pltpu.sync_copy(src_hbm.at[w, pl.ds(h * K2, K2)], src_v)
        pltpu.sync_copy(dst_hbm.at[w, pl.ds(h * K2, K2)], dst_v)
        pltpu.async_copy(x_hbm.at[src_v.at[0]], rows0_v, sem0)

        def body(i, carry):
            j = 2 * i
            pltpu.make_async_copy(x_hbm.at[src_v.at[j]], rows0_v, sem0).wait()
            pltpu.async_copy(x_hbm.at[src_v.at[j + 1]], rows1_v, sem1)
            pltpu.sync_copy(rows0_v, acc.at[dst_v.at[j]], add=True)
            pltpu.make_async_copy(x_hbm.at[src_v.at[j + 1]], rows1_v, sem1).wait()

            @pl.when(j + 2 < K2)
            def _():
                pltpu.async_copy(x_hbm.at[src_v.at[j + 2]], rows0_v, sem0)

            pltpu.sync_copy(rows1_v, acc.at[dst_v.at[j + 1]], add=True)
            return carry

        lax.fori_loop(0, K2 // 2, body, 0)
    plsc.subcore_barrier()
    pltpu.sync_copy(acc.at[pl.ds(s * RPT, RPT)],
                    out_hbm.at[pl.ds(c * NPAD + s * RPT, RPT)])


BN_ROWS = 1024
_GRID = NPAD // BN_ROWS


def _mlp_body(x_ref, p0_ref, p1_ref, w0_ref, b0_ref, w1_ref, b1_ref, o_ref):
    h = x_ref[...] + p0_ref[...] + p1_ref[...]
    h = jnp.dot(h, w0_ref[...], preferred_element_type=jnp.float32) + b0_ref[...]
    h = jnp.maximum(h, 0.0)
    h = jnp.dot(h, w1_ref[...], preferred_element_type=jnp.float32) + b1_ref[...]
    o_ref[...] = jnp.maximum(h, 0.0)


_mlp_call = pl.pallas_call(
    _mlp_body,
    grid=(_GRID,),
    in_specs=[
        pl.BlockSpec((BN_ROWS, D), lambda i: (i, 0)),
        pl.BlockSpec((BN_ROWS, D), lambda i: (i, 0)),
        pl.BlockSpec((BN_ROWS, D), lambda i: (i + _GRID, 0)),
        pl.BlockSpec((D, H), lambda i: (0, 0)),
        pl.BlockSpec((1, H), lambda i: (0, 0)),
        pl.BlockSpec((H, H), lambda i: (0, 0)),
        pl.BlockSpec((1, H), lambda i: (0, 0)),
    ],
    out_specs=pl.BlockSpec((BN_ROWS, H), lambda i: (i, 0)),
    out_shape=jax.ShapeDtypeStruct((NPAD, H), jnp.float32),
)


def _head_body(x1_ref, x2_ref, x3_ref, wa_ref, wb_ref, wc_ref, b1_ref,
               w2_ref, b2_ref, wo_ref, bo_ref, o_ref):
    h = (jnp.dot(x1_ref[...], wa_ref[...], preferred_element_type=jnp.float32)
         + jnp.dot(x2_ref[...], wb_ref[...], preferred_element_type=jnp.float32)
         + jnp.dot(x3_ref[...], wc_ref[...], preferred_element_type=jnp.float32)
         + b1_ref[...])
    h = jnp.maximum(h, 0.0)
    h = jnp.dot(h, w2_ref[...], preferred_element_type=jnp.float32) + b2_ref[...]
    h = jnp.maximum(h, 0.0)
    o = jnp.dot(h, wo_ref[...], preferred_element_type=jnp.float32) + bo_ref[...]
    o_ref[...] = jnp.maximum(o, 0.0)


_head_call = pl.pallas_call(
    _head_body,
    grid=(_GRID,),
    in_specs=[
        pl.BlockSpec((BN_ROWS, H), lambda i: (i, 0)),
        pl.BlockSpec((BN_ROWS, H), lambda i: (i, 0)),
        pl.BlockSpec((BN_ROWS, H), lambda i: (i, 0)),
        pl.BlockSpec((H, 3 * H), lambda i: (0, 0)),
        pl.BlockSpec((H, 3 * H), lambda i: (0, 0)),
        pl.BlockSpec((H, 3 * H), lambda i: (0, 0)),
        pl.BlockSpec((1, 3 * H), lambda i: (0, 0)),
        pl.BlockSpec((3 * H, 3 * H), lambda i: (0, 0)),
        pl.BlockSpec((1, 3 * H), lambda i: (0, 0)),
        pl.BlockSpec((3 * H, 128), lambda i: (0, 0)),
        pl.BlockSpec((1, 128), lambda i: (0, 0)),
    ],
    out_specs=pl.BlockSpec((BN_ROWS, 128), lambda i: (i, 0)),
    out_shape=jax.ShapeDtypeStruct((NPAD, 128), jnp.float32),
)


def _fold_bn(Wl, bl, g, be):
    sc = g * (1.0 / np.sqrt(1.0 + BN_EPS))
    return Wl * sc[None, :], bl * sc + be


def kernel(x, edge_index, batch,
           c1_W0, c1_b0, c1_g0, c1_be0, c1_W1, c1_b1, c1_g1, c1_be1,
           c2_W0, c2_b0, c2_g0, c2_be0, c2_W1, c2_b1, c2_g1, c2_be1,
           c3_W0, c3_b0, c3_g0, c3_be0, c3_W1, c3_b1, c3_g1, c3_be1,
           lin1_W, lin1_b, lin2_W, lin2_b, out_W, out_b):
    del batch
    layers = []
    for (W0, b0, g0, be0, W1, b1, g1, be1) in (
        (c1_W0, c1_b0, c1_g0, c1_be0, c1_W1, c1_b1, c1_g1, c1_be1),
        (c2_W0, c2_b0, c2_g0, c2_be0, c2_W1, c2_b1, c2_g1, c2_be1),
        (c3_W0, c3_b0, c3_g0, c3_be0, c3_W1, c3_b1, c3_g1, c3_be1),
    ):
        W0f, b0f = _fold_bn(W0, b0, g0, be0)
        W1f, b1f = _fold_bn(W1, b1, g1, be1)
        layers.append((W0f, b0f.reshape(1, H), W1f, b1f.reshape(1, H)))

    x_pad = jnp.pad(x, ((0, NPAD - N), (0, 0)))
    junk = N + jnp.arange(EPAD - E, dtype=jnp.int32) % (NPAD - N)
    src = jnp.concatenate([edge_index[0], junk]).reshape(NW, K, CH)
    dst = jnp.concatenate([edge_index[1], junk]).reshape(NW, K, CH)

    h = x_pad
    feats = []
    for (W0f, b0f, W1f, b1f) in layers:
        parts = _segsum_sc(h, src, dst)
        h = _mlp_call(h, parts, parts, W0f, b0f, W1f, b1f)
        feats.append(h)

    x1, x2, x3 = feats
    wa, wb, wc = lin1_W[:H], lin1_W[H:2 * H], lin1_W[2 * H:]
    wo = jnp.pad(out_W, ((0, 0), (0, 127)))
    bo = jnp.pad(out_b, (0, 127)).reshape(1, 128)
    out = _head_call(x1, x2, x3, wa, wb, wc, lin1_b.reshape(1, 3 * H),
                     lin2_W, lin2_b.reshape(1, 3 * H), wo, bo)
    return out[:N, :1]


# speedup vs baseline: 1.0650x; 1.0147x over previous
import functools

import jax
import jax.numpy as jnp
import numpy as np
from jax import lax
from jax.experimental import pallas as pl
from jax.experimental.pallas import tpu as pltpu
from jax.experimental.pallas import tpu_sc as plsc

N = 10000
E = 320000
D = 128
H = 128
BN_EPS = 1e-5

NC = 2
NS = 16
NW = NC * NS
CH = 128
K = 80
EPAD = NW * K * CH
NPAD = 10240
RPT = NPAD // NS

_mesh = plsc.VectorSubcoreMesh(core_axis_name="c", subcore_axis_name="s")


@functools.partial(
    pl.kernel,
    out_type=jax.ShapeDtypeStruct((2 * NPAD, D), jnp.float32),
    mesh=_mesh,
    scratch_types=[
        pltpu.VMEM((K // 2, CH), jnp.int32),
        pltpu.VMEM((K // 2, CH), jnp.int32),
        pltpu.VMEM((CH, D), jnp.float32),
        pltpu.VMEM((CH, D), jnp.float32),
        pltpu.VMEM_SHARED((NPAD, D), jnp.float32),
        pltpu.SemaphoreType.DMA,
        pltpu.SemaphoreType.DMA,
    ],
)
def _segsum_sc(x_hbm, src_hbm, dst_hbm, out_hbm,
               src_v, dst_v, rows0_v, rows1_v, acc, sem0, sem1):
    c = lax.axis_index("c")
    s = lax.axis_index("s")
    w = s * NC + c

    def zbody(r, carry):
        for q in range(D // 16):
            rows0_v[r, pl.ds(q * 16, 16)] = jnp.zeros((16,), jnp.float32)
        return carry

    lax.fori_loop(0, CH, zbody, 0)
    for t in range(RPT // CH):
        pltpu.async_copy(rows0_v, acc.at[pl.ds(s * RPT + t * CH, CH)], sem0)
    for t in range(RPT // CH):
        pltpu.make_async_copy(
            rows0_v, acc.at[pl.ds(s * RPT + t * CH, CH)], sem0).wait()
    plsc.subcore_barrier()

    K2 = K // 2
    for h in range(2):
        pltpu.sync_copy(src_hbm.at[w, pl.ds(h * K2, K2)], src_v)
        pltpu.sync_copy(dst_hbm.at[w, pl.ds(h * K2, K2)], dst_v)
        pltpu.async_copy(x_hbm.at[src_v.at[0]], rows0_v, sem0)

        def body(i, carry):
            j = 2 * i
            pltpu.make_async_copy(x_hbm.at[src_v.at[j]], rows0_v, sem0).wait()
            pltpu.async_copy(x_hbm.at[src_v.at[j + 1]], rows1_v, sem1)
            pltpu.sync_copy(rows0_v, acc.at[dst_v.at[j]], add=True)
            pltpu.make_async_copy(x_hbm.at[src_v.at[j + 1]], rows1_v, sem1).wait()

            @pl.when(j + 2 < K2)
            def _():
                pltpu.async_copy(x_hbm.at[src_v.at[j + 2]], rows0_v, sem0)

            pltpu.sync_copy(rows1_v, acc.at[dst_v.at[j + 1]], add=True)
            return carry

        lax.fori_loop(0, K2 // 2, body, 0)
    plsc.subcore_barrier()
    pltpu.sync_copy(acc.at[pl.ds(s * RPT, RPT)],
                    out_hbm.at[pl.ds(c * NPAD + s * RPT, RPT)])


BN_ROWS = 2048
_GRID = NPAD // BN_ROWS


def _mlp_body(x_ref, p0_ref, p1_ref, w0_ref, b0_ref, w1_ref, b1_ref, o_ref):
    h = x_ref[...] + p0_ref[...] + p1_ref[...]
    h = jnp.dot(h, w0_ref[...], preferred_element_type=jnp.float32) + b0_ref[...]
    h = jnp.maximum(h, 0.0)
    h = jnp.dot(h, w1_ref[...], preferred_element_type=jnp.float32) + b1_ref[...]
    o_ref[...] = jnp.maximum(h, 0.0)


_mlp_call = pl.pallas_call(
    _mlp_body,
    grid=(_GRID,),
    in_specs=[
        pl.BlockSpec((BN_ROWS, D), lambda i: (i, 0)),
        pl.BlockSpec((BN_ROWS, D), lambda i: (i, 0)),
        pl.BlockSpec((BN_ROWS, D), lambda i: (i + _GRID, 0)),
        pl.BlockSpec((D, H), lambda i: (0, 0)),
        pl.BlockSpec((1, H), lambda i: (0, 0)),
        pl.BlockSpec((H, H), lambda i: (0, 0)),
        pl.BlockSpec((1, H), lambda i: (0, 0)),
    ],
    out_specs=pl.BlockSpec((BN_ROWS, H), lambda i: (i, 0)),
    out_shape=jax.ShapeDtypeStruct((NPAD, H), jnp.float32),
)


def _head_body(x1_ref, x2_ref, x3_ref, wa_ref, wb_ref, wc_ref, b1_ref,
               w2_ref, b2_ref, wo_ref, bo_ref, o_ref):
    h = (jnp.dot(x1_ref[...], wa_ref[...], preferred_element_type=jnp.float32)
         + jnp.dot(x2_ref[...], wb_ref[...], preferred_element_type=jnp.float32)
         + jnp.dot(x3_ref[...], wc_ref[...], preferred_element_type=jnp.float32)
         + b1_ref[...])
    h = jnp.maximum(h, 0.0)
    h = jnp.dot(h, w2_ref[...], preferred_element_type=jnp.float32) + b2_ref[...]
    h = jnp.maximum(h, 0.0)
    o = jnp.dot(h, wo_ref[...], preferred_element_type=jnp.float32) + bo_ref[...]
    o_ref[...] = jnp.maximum(o, 0.0)


_head_call = pl.pallas_call(
    _head_body,
    grid=(_GRID,),
    in_specs=[
        pl.BlockSpec((BN_ROWS, H), lambda i: (i, 0)),
        pl.BlockSpec((BN_ROWS, H), lambda i: (i, 0)),
        pl.BlockSpec((BN_ROWS, H), lambda i: (i, 0)),
        pl.BlockSpec((H, 3 * H), lambda i: (0, 0)),
        pl.BlockSpec((H, 3 * H), lambda i: (0, 0)),
        pl.BlockSpec((H, 3 * H), lambda i: (0, 0)),
        pl.BlockSpec((1, 3 * H), lambda i: (0, 0)),
        pl.BlockSpec((3 * H, 3 * H), lambda i: (0, 0)),
        pl.BlockSpec((1, 3 * H), lambda i: (0, 0)),
        pl.BlockSpec((3 * H, 128), lambda i: (0, 0)),
        pl.BlockSpec((1, 128), lambda i: (0, 0)),
    ],
    out_specs=pl.BlockSpec((BN_ROWS, 128), lambda i: (i, 0)),
    out_shape=jax.ShapeDtypeStruct((NPAD, 128), jnp.float32),
)


def _fold_bn(Wl, bl, g, be):
    sc = g * (1.0 / np.sqrt(1.0 + BN_EPS))
    return Wl * sc[None, :], bl * sc + be


def kernel(x, edge_index, batch,
           c1_W0, c1_b0, c1_g0, c1_be0, c1_W1, c1_b1, c1_g1, c1_be1,
           c2_W0, c2_b0, c2_g0, c2_be0, c2_W1, c2_b1, c2_g1, c2_be1,
           c3_W0, c3_b0, c3_g0, c3_be0, c3_W1, c3_b1, c3_g1, c3_be1,
           lin1_W, lin1_b, lin2_W, lin2_b, out_W, out_b):
    del batch
    layers = []
    for (W0, b0, g0, be0, W1, b1, g1, be1) in (
        (c1_W0, c1_b0, c1_g0, c1_be0, c1_W1, c1_b1, c1_g1, c1_be1),
        (c2_W0, c2_b0, c2_g0, c2_be0, c2_W1, c2_b1, c2_g1, c2_be1),
        (c3_W0, c3_b0, c3_g0, c3_be0, c3_W1, c3_b1, c3_g1, c3_be1),
    ):
        W0f, b0f = _fold_bn(W0, b0, g0, be0)
        W1f, b1f = _fold_bn(W1, b1, g1, be1)
        layers.append((W0f, b0f.reshape(1, H), W1f, b1f.reshape(1, H)))

    x_pad = jnp.pad(x, ((0, NPAD - N), (0, 0)))
    junk = N + jnp.arange(EPAD - E, dtype=jnp.int32) % (NPAD - N)
    src = jnp.concatenate([edge_index[0], junk]).reshape(NW, K, CH)
    dst = jnp.concatenate([edge_index[1], junk]).reshape(NW, K, CH)

    h = x_pad
    feats = []
    for (W0f, b0f, W1f, b1f) in layers:
        parts = _segsum_sc(h, src, dst)
        h = _mlp_call(h, parts, parts, W0f, b0f, W1f, b1f)
        feats.append(h)

    x1, x2, x3 = feats
    wa, wb, wc = lin1_W[:H], lin1_W[H:2 * H], lin1_W[2 * H:]
    wo = jnp.pad(out_W, ((0, 0), (0, 127)))
    bo = jnp.pad(out_b, (0, 127)).reshape(1, 128)
    out = _head_call(x1, x2, x3, wa, wb, wc, lin1_b.reshape(1, 3 * H),
                     lin2_W, lin2_b.reshape(1, 3 * H), wo, bo)
    return out[:N, :1]
